# baseline (speedup 1.0000x reference)
import functools

import jax
import jax.numpy as jnp
from jax import lax
from jax.experimental import pallas as pl
from jax.experimental.pallas import tpu as pltpu
from jax.experimental.pallas import tpu_sc as plsc

T, V, D, L = 16384, 100000, 128, 64
_NC, _NS = 2, 16
_NW = _NC * _NS
_TH = T // 2
_BPW = _TH // _NW
_CH = _BPW // 128
_BT = 4096


def _sc_body(idx_hbm, table_hbm, out_hbm, idx_v, rows_v, semG, semO):
    wid = lax.axis_index("s") * _NC + lax.axis_index("c")
    base = wid * _BPW
    pltpu.sync_copy(idx_hbm.at[wid], idx_v)
    gathers = [
        pltpu.async_copy(
            table_hbm.at[idx_v.at[j]],
            rows_v.at[pl.ds(j * 128, 128)],
            semG,
        )
        for j in range(_CH)
    ]
    for g in gathers:
        g.wait()
    pltpu.sync_copy(rows_v, out_hbm.at[pl.ds(base, _BPW)])


_sc_gather_half = pl.kernel(
    _sc_body,
    out_type=jax.ShapeDtypeStruct((_TH, D), jnp.float32),
    mesh=plsc.VectorSubcoreMesh(core_axis_name="c", subcore_axis_name="s"),
    scratch_types=[
        pltpu.VMEM((_CH, 128), jnp.int32),
        pltpu.VMEM((_BPW, D), jnp.float32),
        pltpu.SemaphoreType.DMA,
        pltpu.SemaphoreType.DMA,
    ],
)


def _tc_body_a(g_ref, wemb_ref, wt_ref, b0_ref, b1_ref, out_ref):
    acc = jnp.dot(g_ref[...], wt_ref[...], preferred_element_type=jnp.float32)
    out_ref[...] = wemb_ref[...] + acc + b0_ref[...] + b1_ref[...]


def _tc_body_b(g_ref, wemb_ref, wt_ref, b0_ref, b1_ref, prev_ref, out_ref):
    del prev_ref
    acc = jnp.dot(g_ref[...], wt_ref[...], preferred_element_type=jnp.float32)
    out_ref[...] = wemb_ref[...] + acc + b0_ref[...] + b1_ref[...]


_common_specs = [
    pl.BlockSpec((_BT, 2 * L), lambda i: (i, 0)),
    pl.BlockSpec((_BT, D), lambda i: (i, 0)),
    pl.BlockSpec((2 * L, D), lambda i: (0, 0)),
    pl.BlockSpec((1, D), lambda i: (0, 0)),
    pl.BlockSpec((1, D), lambda i: (0, 0)),
]


def _tc_half_a(g, wemb, wt, b0, b1):
    return pl.pallas_call(
        _tc_body_a,
        out_shape=jax.ShapeDtypeStruct((T, D), jnp.float32),
        grid=(_TH // _BT,),
        in_specs=_common_specs,
        out_specs=pl.BlockSpec((_BT, D), lambda i: (i, 0)),
    )(g, wemb, wt, b0, b1)


def _tc_half_b(g, wemb, wt, b0, b1, prev):
    nblk = _TH // _BT
    return pl.pallas_call(
        _tc_body_b,
        out_shape=jax.ShapeDtypeStruct((T, D), jnp.float32),
        grid=(nblk,),
        in_specs=_common_specs + [pl.BlockSpec(memory_space=pl.ANY)],
        out_specs=pl.BlockSpec((_BT, D), lambda i: (i + nblk, 0)),
        input_output_aliases={5: 0},
    )(g, wemb, wt, b0, b1, prev)


def kernel(sentence_data, batch_sizes, gazetteers_data, word_table, W0, b0, W1, b1):
    del batch_sizes
    idx = sentence_data.reshape(2, _NW, _CH, 128)
    wt = jnp.concatenate([W0.T, W1.T], axis=0)
    b0r, b1r = b0[None, :], b1[None, :]

    wemb_a = _sc_gather_half(idx[0], word_table)
    wemb_b = _sc_gather_half(idx[1], word_table)
    out_a = _tc_half_a(gazetteers_data[:_TH], wemb_a, wt, b0r, b1r)
    return _tc_half_b(gazetteers_data[_TH:], wemb_b, wt, b0r, b1r, out_a)


# speedup vs baseline: 1.2018x; 1.2018x over previous
import functools

import jax
import jax.numpy as jnp
from jax import lax
from jax.experimental import pallas as pl
from jax.experimental.pallas import tpu as pltpu
from jax.experimental.pallas import tpu_sc as plsc

T, V, D, L = 16384, 100000, 128, 64
_NC, _NS = 2, 16
_NW = _NC * _NS
_BPW = T // _NW
_CH = _BPW // 128


def _sc_body(idx_hbm, table_hbm, out_hbm, idx_v, rows_v, sem):
    wid = lax.axis_index("s") * _NC + lax.axis_index("c")
    base = wid * _BPW
    pltpu.sync_copy(idx_hbm.at[wid], idx_v)
    copies = [
        pltpu.async_copy(
            table_hbm.at[idx_v.at[j]],
            rows_v.at[pl.ds(j * 128, 128)],
            sem,
        )
        for j in range(_CH)
    ]
    for c in copies:
        c.wait()
    pltpu.sync_copy(rows_v, out_hbm.at[pl.ds(base, _BPW)])


_sc_gather = pl.kernel(
    _sc_body,
    out_type=jax.ShapeDtypeStruct((T, D), jnp.float32),
    mesh=plsc.VectorSubcoreMesh(core_axis_name="c", subcore_axis_name="s"),
    scratch_types=[
        pltpu.VMEM((_CH, 128), jnp.int32),
        pltpu.VMEM((_BPW, D), jnp.float32),
        pltpu.SemaphoreType.DMA,
    ],
)


def _tc_body(g_ref, wemb_ref, wt_ref, b0_ref, b1_ref, out_ref):
    acc = jnp.dot(g_ref[...], wt_ref[...], preferred_element_type=jnp.float32)
    out_ref[...] = wemb_ref[...] + acc + b0_ref[...] + b1_ref[...]


def _tc_matmul_add(g, wemb, wt, b0, b1):
    bT = 4096
    return pl.pallas_call(
        _tc_body,
        out_shape=jax.ShapeDtypeStruct((T, D), jnp.float32),
        grid=(T // bT,),
        in_specs=[
            pl.BlockSpec((bT, 2 * L), lambda i: (i, 0)),
            pl.BlockSpec((bT, D), lambda i: (i, 0)),
            pl.BlockSpec((2 * L, D), lambda i: (0, 0)),
            pl.BlockSpec((1, D), lambda i: (0, 0)),
            pl.BlockSpec((1, D), lambda i: (0, 0)),
        ],
        out_specs=pl.BlockSpec((bT, D), lambda i: (i, 0)),
    )(g, wemb, wt, b0, b1)


def kernel(sentence_data, batch_sizes, gazetteers_data, word_table, W0, b0, W1, b1):
    del batch_sizes
    idx = sentence_data.reshape(_NW, _CH, 128)
    wemb = _sc_gather(idx, word_table)
    wt = jnp.concatenate([W0.T, W1.T], axis=0)
    return _tc_matmul_add(gazetteers_data, wemb, wt, b0[None, :], b1[None, :])
